# baseline (speedup 1.0000x reference)
import jax
import jax.numpy as jnp
from jax.experimental import pallas as pl
from jax.experimental.pallas import tpu as pltpu

_TM = 512


def _body(x_ref, w_ref, b_ref, o_ref):
    wm = (w_ref[0] + w_ref[1] + w_ref[2] + w_ref[3]) * 0.25
    bm = (b_ref[0] + b_ref[1] + b_ref[2] + b_ref[3]) * 0.25
    o_ref[...] = (
        jnp.dot(x_ref[...], wm, preferred_element_type=jnp.float32) + bm
    )


def kernel(x, W, b):
    T, D = x.shape
    R, _, E = W.shape
    return pl.pallas_call(
        _body,
        grid=(T // _TM,),
        in_specs=[
            pl.BlockSpec((_TM, D), lambda i: (i, 0)),
            pl.BlockSpec((R, D, E), lambda i: (0, 0, 0)),
            pl.BlockSpec((R, E), lambda i: (0, 0)),
        ],
        out_specs=pl.BlockSpec((_TM, E), lambda i: (i, 0)),
        out_shape=jax.ShapeDtypeStruct((T, E), jnp.float32),
        compiler_params=pltpu.CompilerParams(
            dimension_semantics=("arbitrary",),
        ),
    )(x, W, b)


# speedup vs baseline: 1.0072x; 1.0072x over previous
import jax
import jax.numpy as jnp
from jax.experimental import pallas as pl
from jax.experimental.pallas import tpu as pltpu

_TM = 1024


def _body(x_ref, w_ref, b_ref, o_ref):
    wm = (w_ref[0] + w_ref[1] + w_ref[2] + w_ref[3]) * 0.25
    bm = (b_ref[0] + b_ref[1] + b_ref[2] + b_ref[3]) * 0.25
    o_ref[...] = (
        jnp.dot(x_ref[...], wm, preferred_element_type=jnp.float32) + bm
    )


def kernel(x, W, b):
    T, D = x.shape
    R, _, E = W.shape
    return pl.pallas_call(
        _body,
        grid=(T // _TM,),
        in_specs=[
            pl.BlockSpec((_TM, D), lambda i: (i, 0)),
            pl.BlockSpec((R, D, E), lambda i: (0, 0, 0)),
            pl.BlockSpec((R, E), lambda i: (0, 0)),
        ],
        out_specs=pl.BlockSpec((_TM, E), lambda i: (i, 0)),
        out_shape=jax.ShapeDtypeStruct((T, E), jnp.float32),
        compiler_params=pltpu.CompilerParams(
            dimension_semantics=("parallel",),
        ),
    )(x, W, b)
